# Initial kernel scaffold; baseline (speedup 1.0000x reference)
import jax
import jax.numpy as jnp
from jax.experimental import pallas as pl


def kernel(x, edge_index, w1, b1):
    raise NotImplementedError("write your pallas kernel here")


# speedup vs baseline: 14.4938x; 14.4938x over previous
import functools

import jax
import jax.numpy as jnp
from jax import lax
from jax.experimental import pallas as pl
from jax.experimental.pallas import tpu as pltpu
from jax.experimental.pallas import tpu_sc as plsc

N = 10000
E = 320000
D = 128

NC = 2
NT = 16
CHUNK = 128
EPT = 10240
NBLK = EPT // CHUNK
EP = EPT * NT * NC
NA = N + 8

_MESH = plsc.VectorSubcoreMesh(core_axis_name="c", subcore_axis_name="s")


@functools.partial(
    pl.kernel,
    mesh=_MESH,
    out_type=jax.ShapeDtypeStruct((NC, N, D), jnp.float32),
    scratch_types=[
        pltpu.VMEM((NBLK, CHUNK), jnp.int32),
        pltpu.VMEM((CHUNK, D), jnp.float32),
        pltpu.VMEM_SHARED((NA, D), jnp.float32),
    ],
)
def _deg_pass(dst2_hbm, ones_hbm, zeros_hbm, degp_hbm, idx_d, ones_v, acc_sh):
    cid = lax.axis_index("c")
    sid = lax.axis_index("s")
    wid = cid * NT + sid

    pltpu.sync_copy(dst2_hbm.at[pl.ds(wid * NBLK, NBLK)], idx_d)
    pltpu.sync_copy(ones_hbm, ones_v)

    @pl.when(sid < 10)
    def _():
        r = sid * 1000
        pltpu.sync_copy(zeros_hbm.at[pl.ds(r, 1000)], acc_sh.at[pl.ds(r, 1000)])

    plsc.subcore_barrier()

    def body(j, carry):
        pltpu.sync_copy(ones_v, acc_sh.at[idx_d.at[j]], add=True)
        return carry

    lax.fori_loop(0, NBLK, body, 0)
    plsc.subcore_barrier()

    @pl.when(sid < 10)
    def _():
        r = sid * 1000
        pltpu.sync_copy(acc_sh.at[pl.ds(r, 1000)],
                        degp_hbm.at[cid, pl.ds(r, 1000)])


@functools.partial(
    pl.kernel,
    mesh=_MESH,
    out_type=jax.ShapeDtypeStruct((NC, N, D), jnp.float32),
    scratch_types=[
        pltpu.VMEM((2, CHUNK), jnp.int32),
        pltpu.VMEM((NBLK, CHUNK), jnp.int32),
        pltpu.VMEM((2, CHUNK, D), jnp.float32),
        pltpu.VMEM_SHARED((NA, D), jnp.float32),
        pltpu.SemaphoreType.DMA,
        pltpu.SemaphoreType.DMA,
    ],
)
def _agg_pass(src1_hbm, dst2_hbm, g_hbm, zeros_hbm, accp_hbm,
              idx_s, idx_d, rows, acc_sh, sem0, sem1):
    cid = lax.axis_index("c")
    sid = lax.axis_index("s")
    wid = cid * NT + sid
    sems = (sem0, sem1)
    ebase = wid * EPT

    pltpu.sync_copy(dst2_hbm.at[pl.ds(wid * NBLK, NBLK)], idx_d)

    @pl.when(sid < 10)
    def _():
        r = sid * 1000
        pltpu.sync_copy(zeros_hbm.at[pl.ds(r, 1000)], acc_sh.at[pl.ds(r, 1000)])

    plsc.subcore_barrier()

    for b in range(2):
        pltpu.sync_copy(src1_hbm.at[pl.ds(ebase + b * CHUNK, CHUNK)], idx_s.at[b])
        pltpu.async_copy(g_hbm.at[idx_s.at[b]], rows.at[b], sems[b])

    def body(j, carry):
        for b in range(2):
            gi = j * 2 + b
            pltpu.make_async_copy(g_hbm.at[idx_s.at[b]], rows.at[b], sems[b]).wait()
            pltpu.sync_copy(rows.at[b], acc_sh.at[idx_d.at[gi]], add=True)
            pltpu.sync_copy(src1_hbm.at[pl.ds(ebase + (gi + 2) * CHUNK, CHUNK)],
                            idx_s.at[b])
            pltpu.async_copy(g_hbm.at[idx_s.at[b]], rows.at[b], sems[b])
        return carry

    lax.fori_loop(0, NBLK // 2 - 1, body, 0)

    for b in range(2):
        pltpu.make_async_copy(g_hbm.at[idx_s.at[b]], rows.at[b], sems[b]).wait()
        gi = NBLK - 2 + b
        pltpu.sync_copy(rows.at[b], acc_sh.at[idx_d.at[gi]], add=True)

    plsc.subcore_barrier()

    @pl.when(sid < 10)
    def _():
        r = sid * 1000
        pltpu.sync_copy(acc_sh.at[pl.ds(r, 1000)],
                        accp_hbm.at[cid, pl.ds(r, 1000)])


_RB = 2000


def _mm_body(x_ref, w_ref, degp_ref, g_ref, dinv_ref):
    h = jnp.dot(x_ref[...], w_ref[...], preferred_element_type=jnp.float32)
    deg = degp_ref[0, :, 0:1] + degp_ref[1, :, 0:1] + 1.0
    dinv = lax.rsqrt(deg)
    g_ref[...] = h * dinv
    dinv_ref[...] = dinv


def _fin_body(accp_ref, g_ref, dinv_ref, b_ref, o_ref):
    acc = accp_ref[0] + accp_ref[1] + g_ref[...]
    o_ref[...] = jnp.maximum(acc * dinv_ref[...] + b_ref[...], 0.0)


def kernel(x, edge_index, w1, b1):
    src = edge_index[0]
    dst = edge_index[1]
    pad = EP - E
    src1 = jnp.concatenate([src, jnp.zeros((pad,), jnp.int32)])
    dst2 = jnp.concatenate([dst, jnp.full((pad,), N, jnp.int32)]).reshape(-1, CHUNK)
    ones = jnp.ones((CHUNK, D), jnp.float32)
    zeros = jnp.zeros((N, D), jnp.float32)

    degp = _deg_pass(dst2, ones, zeros)

    g, dinv = pl.pallas_call(
        _mm_body,
        grid=(N // _RB,),
        in_specs=[
            pl.BlockSpec((_RB, D), lambda i: (i, 0)),
            pl.BlockSpec((D, D), lambda i: (0, 0)),
            pl.BlockSpec((NC, _RB, D), lambda i: (0, i, 0)),
        ],
        out_specs=[
            pl.BlockSpec((_RB, D), lambda i: (i, 0)),
            pl.BlockSpec((_RB, 1), lambda i: (i, 0)),
        ],
        out_shape=[
            jax.ShapeDtypeStruct((N, D), jnp.float32),
            jax.ShapeDtypeStruct((N, 1), jnp.float32),
        ],
    )(x, w1, degp)

    accp = _agg_pass(src1, dst2, g, zeros)

    out = pl.pallas_call(
        _fin_body,
        grid=(N // _RB,),
        in_specs=[
            pl.BlockSpec((NC, _RB, D), lambda i: (0, i, 0)),
            pl.BlockSpec((_RB, D), lambda i: (i, 0)),
            pl.BlockSpec((_RB, 1), lambda i: (i, 0)),
            pl.BlockSpec((1, D), lambda i: (0, 0)),
        ],
        out_specs=pl.BlockSpec((_RB, D), lambda i: (i, 0)),
        out_shape=jax.ShapeDtypeStruct((N, D), jnp.float32),
    )(accp, g, dinv, b1.reshape(1, D))

    return out


# speedup vs baseline: 31.1039x; 2.1460x over previous
import functools

import jax
import jax.numpy as jnp
from jax import lax
from jax.experimental import pallas as pl
from jax.experimental.pallas import tpu as pltpu
from jax.experimental.pallas import tpu_sc as plsc

N = 10000
E = 320000
D = 128

NC = 2
NT = 16
CHUNK = 128
EPT = 10240
NBLK = EPT // CHUNK
EP = EPT * NT * NC
NA = N + 8

_MESH = plsc.VectorSubcoreMesh(core_axis_name="c", subcore_axis_name="s")


@functools.partial(
    pl.kernel,
    mesh=_MESH,
    out_type=jax.ShapeDtypeStruct((NC, N, D), jnp.float32),
    scratch_types=[
        pltpu.VMEM((NBLK, CHUNK), jnp.int32),
        pltpu.VMEM((CHUNK, D), jnp.float32),
        pltpu.VMEM_SHARED((NA, D), jnp.float32),
    ],
)
def _deg_pass(dst2_hbm, ones_hbm, zeros_hbm, degp_hbm, idx_d, ones_v, acc_sh):
    cid = lax.axis_index("c")
    sid = lax.axis_index("s")
    wid = cid * NT + sid

    pltpu.sync_copy(dst2_hbm.at[pl.ds(wid * NBLK, NBLK)], idx_d)
    pltpu.sync_copy(ones_hbm, ones_v)

    @pl.when(sid < 10)
    def _():
        r = sid * 1000
        pltpu.sync_copy(zeros_hbm.at[pl.ds(r, 1000)], acc_sh.at[pl.ds(r, 1000)])

    plsc.subcore_barrier()

    def body(j, carry):
        pltpu.sync_copy(ones_v, acc_sh.at[idx_d.at[j]], add=True)
        return carry

    lax.fori_loop(0, NBLK, body, 0)
    plsc.subcore_barrier()

    @pl.when(sid < 10)
    def _():
        r = sid * 1000
        pltpu.sync_copy(acc_sh.at[pl.ds(r, 1000)],
                        degp_hbm.at[cid, pl.ds(r, 1000)])


@functools.partial(
    pl.kernel,
    mesh=_MESH,
    out_type=jax.ShapeDtypeStruct((NC, N, D), jnp.float32),
    scratch_types=[
        pltpu.VMEM((2, CHUNK), jnp.int32),
        pltpu.VMEM((NBLK, CHUNK), jnp.int32),
        pltpu.VMEM((2, CHUNK, D), jnp.float32),
        pltpu.VMEM_SHARED((NA, D), jnp.float32),
        pltpu.SemaphoreType.DMA,
        pltpu.SemaphoreType.DMA,
    ],
)
def _agg_pass(src1_hbm, dst2_hbm, g_hbm, zeros_hbm, accp_hbm,
              idx_s, idx_d, rows, acc_sh, sem0, sem1):
    cid = lax.axis_index("c")
    sid = lax.axis_index("s")
    wid = cid * NT + sid
    sems = (sem0, sem1)
    ebase = wid * EPT

    pltpu.sync_copy(dst2_hbm.at[pl.ds(wid * NBLK, NBLK)], idx_d)

    @pl.when(sid < 10)
    def _():
        r = sid * 1000
        pltpu.sync_copy(zeros_hbm.at[pl.ds(r, 1000)], acc_sh.at[pl.ds(r, 1000)])

    plsc.subcore_barrier()

    for b in range(2):
        pltpu.sync_copy(src1_hbm.at[pl.ds(ebase + b * CHUNK, CHUNK)], idx_s.at[b])
        pltpu.async_copy(g_hbm.at[idx_s.at[b]], rows.at[b], sems[b])

    def body(j, carry):
        for b in range(2):
            gi = j * 2 + b
            pltpu.make_async_copy(g_hbm.at[idx_s.at[b]], rows.at[b], sems[b]).wait()
            pltpu.sync_copy(rows.at[b], acc_sh.at[idx_d.at[gi]], add=True)
            pltpu.sync_copy(src1_hbm.at[pl.ds(ebase + (gi + 2) * CHUNK, CHUNK)],
                            idx_s.at[b])
            pltpu.async_copy(g_hbm.at[idx_s.at[b]], rows.at[b], sems[b])
        return carry

    lax.fori_loop(0, NBLK // 2 - 1, body, 0)

    for b in range(2):
        pltpu.make_async_copy(g_hbm.at[idx_s.at[b]], rows.at[b], sems[b]).wait()
        gi = NBLK - 2 + b
        pltpu.sync_copy(rows.at[b], acc_sh.at[idx_d.at[gi]], add=True)

    plsc.subcore_barrier()

    @pl.when(sid < 10)
    def _():
        r = sid * 1000
        pltpu.sync_copy(acc_sh.at[pl.ds(r, 1000)],
                        accp_hbm.at[cid, pl.ds(r, 1000)])


_RB = 2000


def _mm_body(x_ref, w_ref, degp_ref, g_ref, dinv_ref):
    h = jnp.dot(x_ref[...], w_ref[...], preferred_element_type=jnp.float32)
    deg = degp_ref[0, :, 0:1] + degp_ref[1, :, 0:1] + 1.0
    dinv = lax.rsqrt(deg)
    g_ref[...] = h * dinv
    dinv_ref[...] = dinv


def _fin_body(accp_ref, g_ref, dinv_ref, b_ref, o_ref):
    acc = accp_ref[0] + accp_ref[1] + g_ref[...]
    o_ref[...] = jnp.maximum(acc * dinv_ref[...] + b_ref[...], 0.0)


def kernel(x, edge_index, w1, b1):
    src = edge_index[0]
    dst = edge_index[1]
    pad = EP - E
    src1 = jnp.concatenate([src, jnp.arange(pad, dtype=jnp.int32)])
    dst2 = jnp.concatenate([dst, jnp.full((pad,), N, jnp.int32)]).reshape(-1, CHUNK)
    ones = jnp.ones((CHUNK, D), jnp.float32)
    zeros = jnp.zeros((N, D), jnp.float32)

    degp = _deg_pass(dst2, ones, zeros)

    g, dinv = pl.pallas_call(
        _mm_body,
        grid=(N // _RB,),
        in_specs=[
            pl.BlockSpec((_RB, D), lambda i: (i, 0)),
            pl.BlockSpec((D, D), lambda i: (0, 0)),
            pl.BlockSpec((NC, _RB, D), lambda i: (0, i, 0)),
        ],
        out_specs=[
            pl.BlockSpec((_RB, D), lambda i: (i, 0)),
            pl.BlockSpec((_RB, 1), lambda i: (i, 0)),
        ],
        out_shape=[
            jax.ShapeDtypeStruct((N, D), jnp.float32),
            jax.ShapeDtypeStruct((N, 1), jnp.float32),
        ],
    )(x, w1, degp)

    accp = _agg_pass(src1, dst2, g, zeros)

    out = pl.pallas_call(
        _fin_body,
        grid=(N // _RB,),
        in_specs=[
            pl.BlockSpec((NC, _RB, D), lambda i: (0, i, 0)),
            pl.BlockSpec((_RB, D), lambda i: (i, 0)),
            pl.BlockSpec((_RB, 1), lambda i: (i, 0)),
            pl.BlockSpec((1, D), lambda i: (0, 0)),
        ],
        out_specs=pl.BlockSpec((_RB, D), lambda i: (i, 0)),
        out_shape=jax.ShapeDtypeStruct((N, D), jnp.float32),
    )(accp, g, dinv, b1.reshape(1, D))

    return out
